# baseline (speedup 1.0000x reference)
import functools

import jax
import jax.numpy as jnp
from jax import lax
from jax.experimental import pallas as pl
from jax.experimental.pallas import tpu as pltpu
from jax.experimental.pallas import tpu_sc as plsc

_B, _T, _D, _H = 16, 512, 128, 16
_K = 4
_G = _T // _K
_NSLOT = _G + 16


def _emit_body(seq_ref, len_ref, py_ref, w_ref, msum_ref, e_ref):
    f32 = jnp.float32
    py = jnp.clip(py_ref[...], 1e-5, 1.0 - 1e-5)
    p0 = py[:, 0, :]
    p1 = py[:, 1, :]
    l00 = jnp.log1p(-p0)
    l01 = jnp.log(p0)
    l10 = jnp.log1p(-p1)
    l11 = jnp.log(p1)
    a_t = (l01 - l00).T.astype(jnp.bfloat16)
    b_t = (l10 - l00).T.astype(jnp.bfloat16)
    c_t = (l11 - l10 - l01 + l00).T.astype(jnp.bfloat16)
    base = jnp.sum(l00, axis=1)

    for b in range(_B):
        yb = seq_ref[b].astype(jnp.bfloat16)
        ypb = jnp.concatenate([jnp.zeros((1, _D), jnp.bfloat16), yb[:-1]],
                              axis=0)
        eb = (jnp.dot(yb, a_t, preferred_element_type=f32)
              + jnp.dot(ypb, b_t, preferred_element_type=f32)
              + jnp.dot(yb * ypb, c_t, preferred_element_type=f32)
              + base[None, :])
        e_ref[b] = eb

    e_all = e_ref[...]
    m = jnp.max(e_all, axis=2)
    w_ref[...] = jnp.exp(e_all - m[:, :, None])
    tt = lax.broadcasted_iota(jnp.int32, (_B, _T), 1)
    mask = tt < len_ref[...]
    msum_ref[...] = jnp.sum(jnp.where(mask, m, 0.0), axis=1, keepdims=True)


def _sc_body(w_hbm, len_hbm, px_hbm, out_hbm, wv, pv, lv, dv):
    f32 = jnp.float32
    wid = lax.axis_index("s") * 2 + lax.axis_index("c")

    @pl.when(wid < _B)
    def _():
        pltpu.sync_copy(w_hbm.at[wid], wv)
        pltpu.sync_copy(px_hbm, pv)
        pltpu.sync_copy(len_hbm, lv.at[pl.ds(0, _B)])

        lane = lax.broadcasted_iota(jnp.int32, (_H,), 0)
        lenb = lv[pl.ds(wid, _H)][0]
        prows = [jnp.maximum(pv[k], 1e-6) for k in range(_H)]
        idxs = [jnp.full((_H,), k, jnp.int32) for k in range(_H)]
        xors = [lane ^ sh for sh in (1, 2, 4, 8)]
        ones = jnp.ones((_H,), f32)

        def allsum(v):
            for xi in xors:
                v = v + v.at[xi].get(mode="promise_in_bounds")
            return v
        for j in range(_NSLOT // _H):
            dv[pl.ds(j * _H, _H)] = ones
        q0 = jnp.where(lane == 0, 1.0, 0.0).astype(f32)

        def step(t, q):
            w = wv[pl.ds(t * _H, _H)]
            prods = [q.at[idxs[k]].get(mode="promise_in_bounds") * prows[k]
                     for k in range(_H)]
            while len(prods) > 1:
                prods = [prods[j] + prods[j + 1]
                         for j in range(0, len(prods), 2)]
            return prods[0] * w

        ng = lenb // _K

        def group(g, q):
            for i in range(_K):
                q = step(g * _K + i, q)
            zd = allsum(q)
            chunk = dv[pl.ds((g // _H) * _H, _H)]
            dv[pl.ds((g // _H) * _H, _H)] = jnp.where(lane == g % _H, zd,
                                                      chunk)
            return q / zd

        q = lax.fori_loop(0, ng, group, q0)
        q = lax.fori_loop(ng * _K, lenb, step, q)
        dv[pl.ds(_G, _H)] = allsum(q)
        pltpu.sync_copy(dv, out_hbm.at[wid])


def _final_body(msum_ref, sc_ref, out_ref):
    row = sc_ref[...]
    dlog = jnp.sum(jnp.log(row[:, :_G]), axis=1, keepdims=True)
    zflog = jnp.log(row[:, _G:_G + 1])
    out_ref[...] = msum_ref[...] + dlog + zflog


def kernel(sequences, lengths, probs_x, probs_y):
    f32 = jnp.float32
    len2 = lengths.reshape(_B, 1)
    w_flat, msum = pl.pallas_call(
        _emit_body,
        out_shape=[jax.ShapeDtypeStruct((_B, _T, _H), f32),
                   jax.ShapeDtypeStruct((_B, 1), f32)],
        scratch_shapes=[pltpu.VMEM((_B, _T, _H), f32)],
    )(sequences, len2, probs_y)

    sc_kernel = functools.partial(
        pl.kernel,
        out_type=jax.ShapeDtypeStruct((_B, _NSLOT), f32),
        mesh=plsc.VectorSubcoreMesh(core_axis_name="c", subcore_axis_name="s"),
        scratch_types=[
            pltpu.VMEM((_T * _H,), f32),
            pltpu.VMEM((_H, _H), f32),
            pltpu.VMEM((2 * _B,), jnp.int32),
            pltpu.VMEM((_NSLOT,), f32),
        ],
    )(_sc_body)
    scout = sc_kernel(w_flat.reshape(_B, _T * _H), lengths,
                      probs_x.astype(f32))

    out = pl.pallas_call(
        _final_body,
        out_shape=jax.ShapeDtypeStruct((_B, 1), f32),
    )(msum, scout)
    return out.reshape(_B)


# speedup vs baseline: 1.0336x; 1.0336x over previous
import functools

import jax
import jax.numpy as jnp
from jax import lax
from jax.experimental import pallas as pl
from jax.experimental.pallas import tpu as pltpu
from jax.experimental.pallas import tpu_sc as plsc

_B, _T, _D, _H = 16, 512, 128, 16
_K = 4
_G = _T // _K
_NSLOT = _G + 16


def _emit_body(seq_ref, len_ref, py_ref, w_ref, msum_ref, e_ref):
    f32 = jnp.float32
    py = jnp.clip(py_ref[...], 1e-5, 1.0 - 1e-5)
    p0 = py[:, 0, :]
    p1 = py[:, 1, :]
    l00 = jnp.log1p(-p0)
    l01 = jnp.log(p0)
    l10 = jnp.log1p(-p1)
    l11 = jnp.log(p1)
    a_t = (l01 - l00).T.astype(jnp.bfloat16)
    b_t = (l10 - l00).T.astype(jnp.bfloat16)
    c_t = (l11 - l10 - l01 + l00).T.astype(jnp.bfloat16)
    base = jnp.sum(l00, axis=1)

    for b in range(_B):
        yb = seq_ref[b].astype(jnp.bfloat16)
        ypb = jnp.concatenate([jnp.zeros((1, _D), jnp.bfloat16), yb[:-1]],
                              axis=0)
        eb = (jnp.dot(yb, a_t, preferred_element_type=f32)
              + jnp.dot(ypb, b_t, preferred_element_type=f32)
              + jnp.dot(yb * ypb, c_t, preferred_element_type=f32)
              + base[None, :])
        e_ref[b] = eb

    e_all = e_ref[...]
    m = jnp.max(e_all, axis=2)
    w_ref[...] = jnp.exp(e_all - m[:, :, None])
    tt = lax.broadcasted_iota(jnp.int32, (_B, _T), 1)
    mask = tt < len_ref[...]
    msum_ref[...] = jnp.sum(jnp.where(mask, m, 0.0), axis=1, keepdims=True)


def _sc_body(w_hbm, len_hbm, px_hbm, out_hbm, wv, pv, lv, dv):
    f32 = jnp.float32
    wid = lax.axis_index("s")

    @pl.when(wid < _B)
    def _():
        pltpu.sync_copy(w_hbm.at[wid], wv)
        pltpu.sync_copy(px_hbm, pv)
        pltpu.sync_copy(len_hbm, lv.at[pl.ds(0, _B)])

        lane = lax.broadcasted_iota(jnp.int32, (_H,), 0)
        lenb = lv[pl.ds(wid, _H)][0]
        prows = [jnp.maximum(pv[k], 1e-6) for k in range(_H)]
        idxs = [jnp.full((_H,), k, jnp.int32) for k in range(_H)]
        xors = [lane ^ sh for sh in (1, 2, 4, 8)]
        ones = jnp.ones((_H,), f32)

        def allsum(v):
            for xi in xors:
                v = v + v.at[xi].get(mode="promise_in_bounds")
            return v
        for j in range(_NSLOT // _H):
            dv[pl.ds(j * _H, _H)] = ones
        q0 = jnp.where(lane == 0, 1.0, 0.0).astype(f32)

        def step(t, q):
            w = wv[pl.ds(t * _H, _H)]
            prods = [q.at[idxs[k]].get(mode="promise_in_bounds") * prows[k]
                     for k in range(_H)]
            while len(prods) > 1:
                prods = [prods[j] + prods[j + 1]
                         for j in range(0, len(prods), 2)]
            return prods[0] * w

        ng = lenb // _K

        def group(g, q):
            for i in range(_K):
                q = step(g * _K + i, q)
            zd = allsum(q)
            chunk = dv[pl.ds((g // _H) * _H, _H)]
            dv[pl.ds((g // _H) * _H, _H)] = jnp.where(lane == g % _H, zd,
                                                      chunk)
            return q / zd

        q = lax.fori_loop(0, ng, group, q0)
        q = lax.fori_loop(ng * _K, lenb, step, q)
        dv[pl.ds(_G, _H)] = allsum(q)
        pltpu.sync_copy(dv, out_hbm.at[wid])


def _final_body(msum_ref, sc_ref, out_ref):
    row = sc_ref[...]
    dlog = jnp.sum(jnp.log(row[:, :_G]), axis=1, keepdims=True)
    zflog = jnp.log(row[:, _G:_G + 1])
    out_ref[...] = msum_ref[...] + dlog + zflog


def kernel(sequences, lengths, probs_x, probs_y):
    f32 = jnp.float32
    len2 = lengths.reshape(_B, 1)
    w_flat, msum = pl.pallas_call(
        _emit_body,
        out_shape=[jax.ShapeDtypeStruct((_B, _T, _H), f32),
                   jax.ShapeDtypeStruct((_B, 1), f32)],
        scratch_shapes=[pltpu.VMEM((_B, _T, _H), f32)],
    )(sequences, len2, probs_y)

    sc_kernel = functools.partial(
        pl.kernel,
        out_type=jax.ShapeDtypeStruct((_B, _NSLOT), f32),
        mesh=plsc.VectorSubcoreMesh(core_axis_name="c", subcore_axis_name="s",
                                    num_cores=1),
        scratch_types=[
            pltpu.VMEM((_T * _H,), f32),
            pltpu.VMEM((_H, _H), f32),
            pltpu.VMEM((2 * _B,), jnp.int32),
            pltpu.VMEM((_NSLOT,), f32),
        ],
    )(_sc_body)
    scout = sc_kernel(w_flat.reshape(_B, _T * _H), lengths,
                      probs_x.astype(f32))

    out = pl.pallas_call(
        _final_body,
        out_shape=jax.ShapeDtypeStruct((_B, 1), f32),
    )(msum, scout)
    return out.reshape(_B)


# speedup vs baseline: 1.0554x; 1.0211x over previous
import functools

import jax
import jax.numpy as jnp
from jax import lax
from jax.experimental import pallas as pl
from jax.experimental.pallas import tpu as pltpu
from jax.experimental.pallas import tpu_sc as plsc

_B, _T, _D, _H = 16, 512, 128, 16
_K = 4
_G = _T // _K

_LOG2_POLY = (-3.028324974420129, 6.065858861162175, -5.2641555241877205,
              3.2188698138651994, -1.2342798994632271, 0.26686276781534235,
              -0.024825984443686732)
_LN2 = 0.6931471805599453


def _emit_body(seq_ref, len_ref, py_ref, w_ref, msum_ref, e_ref):
    f32 = jnp.float32
    py = jnp.clip(py_ref[...], 1e-5, 1.0 - 1e-5)
    p0 = py[:, 0, :]
    p1 = py[:, 1, :]
    l00 = jnp.log1p(-p0)
    l01 = jnp.log(p0)
    l10 = jnp.log1p(-p1)
    l11 = jnp.log(p1)
    a_t = (l01 - l00).T.astype(jnp.bfloat16)
    b_t = (l10 - l00).T.astype(jnp.bfloat16)
    c_t = (l11 - l10 - l01 + l00).T.astype(jnp.bfloat16)
    base = jnp.sum(l00, axis=1)

    for b in range(_B):
        yb = seq_ref[b].astype(jnp.bfloat16)
        ypb = jnp.concatenate([jnp.zeros((1, _D), jnp.bfloat16), yb[:-1]],
                              axis=0)
        eb = (jnp.dot(yb, a_t, preferred_element_type=f32)
              + jnp.dot(ypb, b_t, preferred_element_type=f32)
              + jnp.dot(yb * ypb, c_t, preferred_element_type=f32)
              + base[None, :])
        e_ref[b] = eb

    e_all = e_ref[...]
    m = jnp.max(e_all, axis=2)
    w_ref[...] = jnp.exp(e_all - m[:, :, None])
    tt = lax.broadcasted_iota(jnp.int32, (_B, _T), 1)
    mask = tt < len_ref[...]
    msum = jnp.sum(jnp.where(mask, m, 0.0), axis=1, keepdims=True)
    msum_ref[...] = jnp.broadcast_to(msum, (_B, _H))


def _vln(x):
    y = x * (2.0 ** 80)
    e = jnp.full((_H,), -80.0, jnp.float32)
    for s in (64, 32, 16, 8, 4, 2, 1):
        big = y >= (2.0 ** s)
        y = jnp.where(big, y * (2.0 ** -s), y)
        e = jnp.where(big, e + s, e)
    p = jnp.full((_H,), _LOG2_POLY[-1], jnp.float32)
    for c in _LOG2_POLY[-2::-1]:
        p = p * y + c
    return (e + p) * _LN2


def _sc_body(w_hbm, len_hbm, px_hbm, ms_hbm, out_hbm, wv, pv, lv, dv, av):
    f32 = jnp.float32
    wid = lax.axis_index("s")

    @pl.when(wid < _B)
    def _():
        pltpu.sync_copy(w_hbm.at[wid], wv)
        pltpu.sync_copy(px_hbm, pv)
        pltpu.sync_copy(len_hbm, lv.at[pl.ds(0, _B)])
        pltpu.sync_copy(ms_hbm.at[wid], av)

        lane = lax.broadcasted_iota(jnp.int32, (_H,), 0)
        lenb = lv[pl.ds(wid, _H)][0]
        prows = [jnp.maximum(pv[k], 1e-6) for k in range(_H)]
        idxs = [jnp.full((_H,), k, jnp.int32) for k in range(_H)]
        xors = [lane ^ sh for sh in (1, 2, 4, 8)]
        ones = jnp.ones((_H,), f32)

        def allsum(v):
            for xi in xors:
                v = v + v.at[xi].get(mode="promise_in_bounds")
            return v
        for j in range(_G // _H):
            dv[pl.ds(j * _H, _H)] = ones
        q0 = jnp.where(lane == 0, 1.0, 0.0).astype(f32)

        def step(t, q):
            w = wv[pl.ds(t * _H, _H)]
            prods = [q.at[idxs[k]].get(mode="promise_in_bounds") * prows[k]
                     for k in range(_H)]
            while len(prods) > 1:
                prods = [prods[j] + prods[j + 1]
                         for j in range(0, len(prods), 2)]
            return prods[0] * w

        ng = lenb // _K

        def group(g, q):
            for i in range(_K):
                q = step(g * _K + i, q)
            zd = allsum(q)
            chunk = dv[pl.ds((g // _H) * _H, _H)]
            dv[pl.ds((g // _H) * _H, _H)] = jnp.where(lane == g % _H, zd,
                                                      chunk)
            return q / zd

        q = lax.fori_loop(0, ng, group, q0)
        q = lax.fori_loop(ng * _K, lenb, step, q)
        zf = allsum(q)

        acc = _vln(dv[pl.ds(0, _H)])
        for j in range(1, _G // _H):
            acc = acc + _vln(dv[pl.ds(j * _H, _H)])
        ans = allsum(acc) + _vln(zf) + av[...]
        av[...] = ans
        pltpu.sync_copy(av, out_hbm.at[wid])


def kernel(sequences, lengths, probs_x, probs_y):
    f32 = jnp.float32
    len2 = lengths.reshape(_B, 1)
    w_flat, msum = pl.pallas_call(
        _emit_body,
        out_shape=[jax.ShapeDtypeStruct((_B, _T, _H), f32),
                   jax.ShapeDtypeStruct((_B, _H), f32)],
        scratch_shapes=[pltpu.VMEM((_B, _T, _H), f32)],
    )(sequences, len2, probs_y)

    sc_kernel = functools.partial(
        pl.kernel,
        out_type=jax.ShapeDtypeStruct((_B, _H), f32),
        mesh=plsc.VectorSubcoreMesh(core_axis_name="c", subcore_axis_name="s",
                                    num_cores=1),
        scratch_types=[
            pltpu.VMEM((_T * _H,), f32),
            pltpu.VMEM((_H, _H), f32),
            pltpu.VMEM((2 * _B,), jnp.int32),
            pltpu.VMEM((_G,), f32),
            pltpu.VMEM((_H,), f32),
        ],
    )(_sc_body)
    scout = sc_kernel(w_flat.reshape(_B, _T * _H), lengths,
                      probs_x.astype(f32), msum)
    return scout[:, 0]
